# baseline (speedup 1.0000x reference)
import functools

import jax
import jax.numpy as jnp
from jax import lax
from jax.experimental import pallas as pl
from jax.experimental.pallas import tpu as pltpu
from jax.experimental.pallas import tpu_sc as plsc

_N = 19
_DELTA = 0.1
_NW = 32
_L = 16
_G0 = 10
_ROWS = 8


def _make_sc(B, C, H, W):
    rows_w = (B * H) // _NW
    nchunks = rows_w // _ROWS
    wpb = _NW // B
    acc_len = 2 * _N * _L
    groups = (tuple(range(_G0)), tuple(range(_G0, C)))
    mesh = plsc.VectorSubcoreMesh(core_axis_name="c", subcore_axis_name="s")

    @functools.partial(
        pl.kernel,
        mesh=mesh,
        compiler_params=pltpu.CompilerParams(
            needs_layout_passes=False, use_tc_tiling_on_sc=True),
        out_type=jax.ShapeDtypeStruct((_NW * acc_len,), jnp.float32),
        scratch_types=[
            pltpu.VMEM((_N * _N,), jnp.float32),
            pltpu.VMEM((_G0, _ROWS, W), jnp.float32),
            pltpu.VMEM((_G0, _ROWS, W), jnp.float32),
            pltpu.VMEM((_ROWS, W), jnp.int32),
            pltpu.VMEM((_ROWS, W), jnp.int32),
            pltpu.VMEM((acc_len,), jnp.float32),
            pltpu.SemaphoreType.DMA,
            pltpu.SemaphoreType.DMA,
            pltpu.SemaphoreType.DMA,
            pltpu.SemaphoreType.DMA,
        ],
    )
    def k(lg_hbm, lab_hbm, pf_hbm, out_hbm,
          pf_v, bufA, bufB, lblA, lblB, acc, semA, semB, lsemA, lsemB):
        wid = lax.axis_index("s") * 2 + lax.axis_index("c")
        b = wid // wpb
        r0 = (wid - b * wpb) * rows_w
        pltpu.sync_copy(pf_hbm, pf_v)
        for t in range(2 * _N):
            acc[pl.ds(t * _L, _L)] = jnp.zeros((_L,), jnp.float32)
        lane = lax.broadcasted_iota(jnp.int32, (_L,), 0)
        ones = jnp.ones((_L,), jnp.float32)

        def ldma(j, g, buf, sem):
            return [
                pltpu.make_async_copy(
                    lg_hbm.at[b * C + c, pl.ds(r0 + j * _ROWS, _ROWS), :],
                    buf.at[ci], sem)
                for ci, c in enumerate(groups[g])
            ]

        def labdma(j, lb, sem):
            return pltpu.make_async_copy(
                lab_hbm.at[pl.ds(b * H + r0 + j * _ROWS, _ROWS), :], lb, sem)

        def start_l(j, g, buf, sem):
            for h in ldma(j, g, buf, sem):
                h.start()

        def drain_l(j, g, buf, sem):
            for h in ldma(j, g, buf, sem):
                h.wait()

        def compute(g, buf, lb, with_counts):
            chans = groups[g]

            for r in range(_ROWS):
                def vec(i, carry):
                    col = i * _L
                    lab16 = lb[r, pl.ds(col, _L)]
                    tbase = lab16 * _N
                    accs = [jnp.zeros((_L,), jnp.float32) for _ in range(4)]
                    for ci, c in enumerate(chans):
                        x = buf[ci, r, pl.ds(col, _L)]
                        m = plsc.load_gather(pf_v, [tbase + c])
                        e = jnp.maximum(jnp.abs(x - m) - _DELTA, 0.0)
                        accs[ci % 4] = accs[ci % 4] + e
                    sidx = lab16 * _L + lane
                    plsc.addupdate_scatter(
                        acc, [sidx], (accs[0] + accs[1]) + (accs[2] + accs[3]))
                    if with_counts:
                        plsc.addupdate_scatter(acc, [_N * _L + sidx], ones)
                    return carry

                lax.fori_loop(0, W // _L, vec, 0)

        start_l(0, 0, bufA, semA)
        labdma(0, lblA, lsemA).start()

        def pair(t, carry):
            j0 = t * 2
            start_l(j0, 1, bufB, semB)
            labdma(j0 + 1, lblB, lsemB).start()
            drain_l(j0, 0, bufA, semA)
            labdma(j0, lblA, lsemA).wait()
            compute(0, bufA, lblA, True)
            start_l(j0 + 1, 0, bufA, semA)
            drain_l(j0, 1, bufB, semB)
            compute(1, bufB, lblA, False)
            start_l(j0 + 1, 1, bufB, semB)
            drain_l(j0 + 1, 0, bufA, semA)
            labdma(j0 + 1, lblB, lsemB).wait()
            compute(0, bufA, lblB, True)

            @pl.when(t + 1 < nchunks // 2)
            def _():
                start_l(j0 + 2, 0, bufA, semA)
                labdma(j0 + 2, lblA, lsemA).start()

            drain_l(j0 + 1, 1, bufB, semB)
            compute(1, bufB, lblB, False)
            return carry

        lax.fori_loop(0, nchunks // 2, pair, 0)
        pltpu.sync_copy(acc, out_hbm.at[pl.ds(wid * acc_len, acc_len)])

    return k


def kernel(logits, sem_gt, is_train, previous_features, previous_count, var):
    del is_train
    B, C, H, W = logits.shape
    out = _make_sc(B, C, H, W)(
        logits.reshape(B * C, H, W),
        sem_gt.reshape(B * H, W),
        previous_features.reshape(-1),
    )
    o = out.reshape(_NW, 2, _N, _L)
    sums = jnp.sum(o[:, 0], axis=(0, 2))
    cnts = jnp.sum(o[:, 1], axis=(0, 2))
    means = sums / jnp.maximum(cnts * C, 1.0)
    valid = (previous_count > 0) & (jnp.sum(var, axis=1) != 0) & (cnts > 0)
    valid = valid.at[0].set(False)
    return jnp.sum(jnp.where(valid, means, 0.0))


# speedup vs baseline: 1.0195x; 1.0195x over previous
import functools

import jax
import jax.numpy as jnp
from jax import lax
from jax.experimental import pallas as pl
from jax.experimental.pallas import tpu as pltpu
from jax.experimental.pallas import tpu_sc as plsc

_N = 19
_DELTA = 0.1
_NW = 32
_L = 16
_G0 = 10
_ROWS = 8


def _make_sc(B, C, H, W):
    rows_w = (B * H) // _NW
    nchunks = rows_w // _ROWS
    wpb = _NW // B
    acc_len = 2 * _N * _L
    groups = (tuple(range(_G0)), tuple(range(_G0, C)))
    mesh = plsc.VectorSubcoreMesh(core_axis_name="c", subcore_axis_name="s")

    @functools.partial(
        pl.kernel,
        mesh=mesh,
        compiler_params=pltpu.CompilerParams(
            needs_layout_passes=False, use_tc_tiling_on_sc=True),
        out_type=jax.ShapeDtypeStruct((_NW * acc_len,), jnp.float32),
        scratch_types=[
            pltpu.VMEM((_N * _N,), jnp.float32),
            pltpu.VMEM((_G0, _ROWS, W), jnp.float32),
            pltpu.VMEM((_G0, _ROWS, W), jnp.float32),
            pltpu.VMEM((_ROWS, W), jnp.int32),
            pltpu.VMEM((_ROWS, W), jnp.int32),
            pltpu.VMEM((acc_len,), jnp.float32),
            pltpu.SemaphoreType.DMA,
            pltpu.SemaphoreType.DMA,
            pltpu.SemaphoreType.DMA,
            pltpu.SemaphoreType.DMA,
        ],
    )
    def k(lg_hbm, lab_hbm, pf_hbm, out_hbm,
          pf_v, bufA, bufB, lblA, lblB, acc, semA, semB, lsemA, lsemB):
        wid = lax.axis_index("s") * 2 + lax.axis_index("c")
        b = wid // wpb
        r0 = (wid - b * wpb) * rows_w
        pltpu.sync_copy(pf_hbm, pf_v)
        for t in range(2 * _N):
            acc[pl.ds(t * _L, _L)] = jnp.zeros((_L,), jnp.float32)
        lane = lax.broadcasted_iota(jnp.int32, (_L,), 0)
        ones = jnp.ones((_L,), jnp.float32)

        def ldma(j, g, buf, sem):
            return [
                pltpu.make_async_copy(
                    lg_hbm.at[b * C + c, pl.ds(r0 + j * _ROWS, _ROWS), :],
                    buf.at[ci], sem)
                for ci, c in enumerate(groups[g])
            ]

        def labdma(j, lb, sem):
            return pltpu.make_async_copy(
                lab_hbm.at[pl.ds(b * H + r0 + j * _ROWS, _ROWS), :], lb, sem)

        def start_l(j, g, buf, sem):
            for h in ldma(j, g, buf, sem):
                h.start()

        def drain_l(j, g, buf, sem):
            for h in ldma(j, g, buf, sem):
                h.wait()

        def compute(g, buf, lb, with_counts):
            chans = groups[g]

            for r in range(_ROWS):
                def vec(i, carry):
                    col = i * _L
                    lab16 = lb[r, pl.ds(col, _L)]
                    tbase = lab16 * _N
                    a0 = jnp.zeros((_L,), jnp.float32)
                    a1 = jnp.zeros((_L,), jnp.float32)
                    for ci, c in enumerate(chans):
                        x = buf[ci, r, pl.ds(col, _L)]
                        m = plsc.load_gather(pf_v, [tbase + c])
                        e = jnp.maximum(jnp.abs(x - m) - _DELTA, 0.0)
                        if ci % 2 == 0:
                            a0 = a0 + e
                        else:
                            a1 = a1 + e
                    sidx = lab16 * _L + lane
                    plsc.addupdate_scatter(acc, [sidx], a0 + a1)
                    if with_counts:
                        plsc.addupdate_scatter(acc, [_N * _L + sidx], ones)
                    return carry

                lax.fori_loop(0, W // _L, vec, 0)

        start_l(0, 0, bufA, semA)
        labdma(0, lblA, lsemA).start()

        def pair(t, carry):
            j0 = t * 2
            start_l(j0, 1, bufB, semB)
            labdma(j0 + 1, lblB, lsemB).start()
            drain_l(j0, 0, bufA, semA)
            labdma(j0, lblA, lsemA).wait()
            compute(0, bufA, lblA, True)
            start_l(j0 + 1, 0, bufA, semA)
            drain_l(j0, 1, bufB, semB)
            compute(1, bufB, lblA, False)
            start_l(j0 + 1, 1, bufB, semB)
            drain_l(j0 + 1, 0, bufA, semA)
            labdma(j0 + 1, lblB, lsemB).wait()
            compute(0, bufA, lblB, True)

            @pl.when(t + 1 < nchunks // 2)
            def _():
                start_l(j0 + 2, 0, bufA, semA)
                labdma(j0 + 2, lblA, lsemA).start()

            drain_l(j0 + 1, 1, bufB, semB)
            compute(1, bufB, lblB, False)
            return carry

        lax.fori_loop(0, nchunks // 2, pair, 0)
        pltpu.sync_copy(acc, out_hbm.at[pl.ds(wid * acc_len, acc_len)])

    return k


def kernel(logits, sem_gt, is_train, previous_features, previous_count, var):
    del is_train
    B, C, H, W = logits.shape
    out = _make_sc(B, C, H, W)(
        logits.reshape(B * C, H, W),
        sem_gt.reshape(B * H, W),
        previous_features.reshape(-1),
    )
    o = out.reshape(_NW, 2, _N, _L)
    sums = jnp.sum(o[:, 0], axis=(0, 2))
    cnts = jnp.sum(o[:, 1], axis=(0, 2))
    means = sums / jnp.maximum(cnts * C, 1.0)
    valid = (previous_count > 0) & (jnp.sum(var, axis=1) != 0) & (cnts > 0)
    valid = valid.at[0].set(False)
    return jnp.sum(jnp.where(valid, means, 0.0))
